# baseline (speedup 1.0000x reference)
import functools

import jax
import jax.numpy as jnp
from jax import lax
from jax.experimental import pallas as pl
from jax.experimental.pallas import tpu as pltpu
from jax.experimental.pallas import tpu_sc as plsc


def _fmt_table_body(in_ref, out_ref):
    a = in_ref[...]
    for t in range(a.shape[1] // 1024):
        b = jnp.concatenate(
            [a[:, 1024 * t + 128 * j:1024 * t + 128 * (j + 1)]
             for j in range(8)], axis=0)
        out_ref[128 * t:128 * (t + 1), :] = b.T


@functools.lru_cache(maxsize=None)
def _make_fmt_table(V1, D):
    blk = 32768
    grid = (pl.cdiv(V1, blk),)
    n_out = grid[0] * (blk * D // 128)
    return pl.pallas_call(
        _fmt_table_body,
        grid=grid,
        in_specs=[pl.BlockSpec((D, blk), lambda i: (0, i))],
        out_specs=pl.BlockSpec((blk * D // 128, 128), lambda i: (i, 0)),
        out_shape=jax.ShapeDtypeStruct((n_out, 128), jnp.float32),
    )


def _fmt_x_body(in_ref, out_ref):
    ids = in_ref[...]
    sig = (ids & ~jnp.int32(1023)) \
        | ((ids & jnp.int32(127)) << 3) \
        | ((ids >> 7) & jnp.int32(7))
    out_ref[...] = sig.reshape(out_ref.shape)


@functools.lru_cache(maxsize=None)
def _make_fmt_x(F, T):
    return pl.pallas_call(
        _fmt_x_body,
        grid=(1,),
        in_specs=[pl.BlockSpec((F, T), lambda i: (0, 0))],
        out_specs=pl.BlockSpec((F * T // 128, 128), lambda i: (0, 0)),
        out_shape=jax.ShapeDtypeStruct((F * T // 128, 128), jnp.int32),
    )


@functools.lru_cache(maxsize=None)
def _make_gather(T, F, D, V_pad, t0, TL):
    NW = 32
    TPW = TL // NW
    CH = TPW
    CPF = TPW // CH
    NCH = F * CPF

    mesh = plsc.VectorSubcoreMesh(core_axis_name="c", subcore_axis_name="s")

    @functools.partial(
        pl.kernel,
        out_type=jax.ShapeDtypeStruct((TL, D), jnp.float32),
        mesh=mesh,
        scratch_types=[
            pltpu.VMEM((TPW * F,), jnp.int32),
            pltpu.VMEM((TPW, D), jnp.float32),
            pltpu.SemaphoreType.DMA,
            pltpu.SemaphoreType.DMA,
        ],
        compiler_params=pltpu.CompilerParams(use_tc_tiling_on_sc=False),
    )
    def gather_kernel(x_hbm, table_hbm, out_hbm, idx_v, emb_v, sem_f, sem_g):
        wid = lax.axis_index("s") * 2 + lax.axis_index("c")
        base_t = wid * TPW

        fills = [
            pltpu.async_copy(
                x_hbm.at[pl.ds(f * T + t0 + base_t, TPW)],
                idx_v.at[pl.ds(f * TPW, TPW)], sem_f)
            for f in range(F)
        ]

        zeros = jnp.zeros((16,), jnp.float32)

        @pl.loop(0, TPW, unroll=8)
        def _(r):
            emb_v[r, :] = zeros

        for cp in fills:
            cp.wait()

        @pl.loop(0, NCH, init_carry=0)
        def _(c, base):
            pltpu.async_copy(
                table_hbm.at[idx_v.at[pl.ds(CH * c, CH)]],
                emb_v.at[pl.ds(base, CH)], sem_g, add=True)
            nxt = base + CH
            return jnp.where(nxt == TPW, 0, nxt)

        for _ in range(F):
            pltpu.make_async_copy(
                table_hbm.at[pl.ds(0, TPW)], emb_v, sem_g).wait()

        pltpu.sync_copy(emb_v, out_hbm.at[pl.ds(base_t, TPW)])

    return gather_kernel


def _mlp_body(v_ref, w1_ref, b1_ref, w2_ref, b2_ref, wo_ref, bo_ref,
              out_ref):
    v = v_ref[...]
    h1b = jnp.maximum(
        jnp.dot(v, w1_ref[...], preferred_element_type=jnp.float32)
        + b1_ref[...], 0.0)
    h1 = h1b.reshape(v.shape[0] * 8, w1_ref.shape[1] // 8)
    h2 = jnp.maximum(
        jnp.dot(h1, w2_ref[...], preferred_element_type=jnp.float32)
        + b2_ref[...], 0.0)
    o = jnp.dot(h2, wo_ref[...], preferred_element_type=jnp.float32) \
        + bo_ref[...]
    p = jax.nn.sigmoid(o)
    pc = jnp.concatenate([p[:, 0:1], p[:, 0:1] * p[:, 1:2]], axis=1)
    out_ref[...] = pc.T


@functools.lru_cache(maxsize=None)
def _make_mlp(T, D, H1c, H2c):
    BT = 3200 if T % 3200 == 0 else 2048
    grid = (T // BT,)

    def full(shape):
        return pl.BlockSpec(shape, lambda i: (0, 0))

    return pl.pallas_call(
        _mlp_body,
        grid=grid,
        in_specs=[
            pl.BlockSpec((BT // 8, 128), lambda i: (i, 0)),
            full((D * 8, H1c * 8)),
            full((1, H1c * 8)),
            full((H1c, H2c)),
            full((1, H2c)),
            full((H2c, 2)),
            full((1, 2)),
        ],
        out_specs=pl.BlockSpec((2, BT), lambda i: (0, i)),
        out_shape=jax.ShapeDtypeStruct((2, T), jnp.float32),
    )


def kernel(x, label_length, table,
           ctr_W1, ctr_b1, ctr_W2, ctr_b2, ctr_Wo, ctr_bo,
           cvr_W1, cvr_b1, cvr_W2, cvr_b2, cvr_Wo, cvr_bo):
    T, F = x.shape
    V1, D = table.shape
    H1 = ctr_W1.shape[1]
    H2 = ctr_W2.shape[1]

    table_fmt = _make_fmt_table(V1, D)(table.T)
    V_pad = table_fmt.shape[0] * 128 // D
    table_lin = table_fmt.reshape(V_pad, D)

    x_fmt = _make_fmt_x(F, T)(x.T)
    x_flat = x_fmt.reshape(F * T)

    NPART = 8
    TL = T // NPART

    z12 = jnp.zeros((H1, H2), jnp.float32)
    z2o = jnp.zeros((H2, 1), jnp.float32)
    W1 = jnp.concatenate([ctr_W1, cvr_W1], axis=1)
    b1 = jnp.concatenate([ctr_b1, cvr_b1])
    W1big = jnp.kron(jnp.eye(8, dtype=jnp.float32), W1)
    b1big = jnp.tile(b1, 8)[None, :]
    W2 = jnp.concatenate(
        [jnp.concatenate([ctr_W2, z12], axis=1),
         jnp.concatenate([z12, cvr_W2], axis=1)], axis=0)
    b2 = jnp.concatenate([ctr_b2, cvr_b2])[None, :]
    Wo = jnp.concatenate(
        [jnp.concatenate([ctr_Wo, z2o], axis=1),
         jnp.concatenate([z2o, cvr_Wo], axis=1)], axis=0)
    bo = jnp.concatenate([ctr_bo, cvr_bo])[None, :]

    outs = []
    for k in range(NPART):
        emb = _make_gather(T, F, D, V_pad, k * TL, TL)(x_flat, table_lin)
        emb2 = emb.reshape(TL * D // 128, 128)
        outs.append(_make_mlp(TL, D, 2 * H1, 2 * H2)(
            emb2, W1big, b1big, W2, b2, Wo, bo))
    return jnp.concatenate(outs, axis=1).T


# speedup vs baseline: 1.1557x; 1.1557x over previous
import functools

import jax
import jax.numpy as jnp
from jax import lax
from jax.experimental import pallas as pl
from jax.experimental.pallas import tpu as pltpu
from jax.experimental.pallas import tpu_sc as plsc


def _fmt_table_body(in_ref, out_ref):
    a = in_ref[...]
    for t in range(a.shape[1] // 1024):
        b = jnp.concatenate(
            [a[:, 1024 * t + 128 * j:1024 * t + 128 * (j + 1)]
             for j in range(8)], axis=0)
        out_ref[128 * t:128 * (t + 1), :] = b.T


@functools.lru_cache(maxsize=None)
def _make_fmt_table(V1, D):
    blk = 32768
    grid = (pl.cdiv(V1, blk),)
    n_out = grid[0] * (blk * D // 128)
    return pl.pallas_call(
        _fmt_table_body,
        grid=grid,
        in_specs=[pl.BlockSpec((D, blk), lambda i: (0, i))],
        out_specs=pl.BlockSpec((blk * D // 128, 128), lambda i: (i, 0)),
        out_shape=jax.ShapeDtypeStruct((n_out, 128), jnp.float32),
    )


def _fmt_x_body(in_ref, out_ref):
    ids = in_ref[...]
    sig = (ids & ~jnp.int32(1023)) \
        | ((ids & jnp.int32(127)) << 3) \
        | ((ids >> 7) & jnp.int32(7))
    out_ref[...] = sig.reshape(out_ref.shape)


@functools.lru_cache(maxsize=None)
def _make_fmt_x(F, T):
    return pl.pallas_call(
        _fmt_x_body,
        grid=(1,),
        in_specs=[pl.BlockSpec((F, T), lambda i: (0, 0))],
        out_specs=pl.BlockSpec((F * T // 128, 128), lambda i: (0, 0)),
        out_shape=jax.ShapeDtypeStruct((F * T // 128, 128), jnp.int32),
    )


@functools.lru_cache(maxsize=None)
def _make_gather(T, F, D, V_pad, t0, TL):
    NW = 32
    TPW = TL // NW
    CH = TPW
    CPF = TPW // CH
    NCH = F * CPF

    mesh = plsc.VectorSubcoreMesh(core_axis_name="c", subcore_axis_name="s")

    @functools.partial(
        pl.kernel,
        out_type=jax.ShapeDtypeStruct((TL, D), jnp.float32),
        mesh=mesh,
        scratch_types=[
            pltpu.VMEM((TPW * F,), jnp.int32),
            pltpu.VMEM((TPW, D), jnp.float32),
            pltpu.SemaphoreType.DMA,
            pltpu.SemaphoreType.DMA,
        ],
        compiler_params=pltpu.CompilerParams(use_tc_tiling_on_sc=False),
    )
    def gather_kernel(x_hbm, table_hbm, out_hbm, idx_v, emb_v, sem_f, sem_g):
        wid = lax.axis_index("s") * 2 + lax.axis_index("c")
        base_t = wid * TPW

        fills = [
            pltpu.async_copy(
                x_hbm.at[pl.ds(f * T + t0 + base_t, TPW)],
                idx_v.at[pl.ds(f * TPW, TPW)], sem_f)
            for f in range(F)
        ]

        zeros = jnp.zeros((16,), jnp.float32)

        @pl.loop(0, TPW, unroll=8)
        def _(r):
            emb_v[r, :] = zeros

        for cp in fills:
            cp.wait()

        @pl.loop(0, NCH, init_carry=0)
        def _(c, base):
            pltpu.async_copy(
                table_hbm.at[idx_v.at[pl.ds(CH * c, CH)]],
                emb_v.at[pl.ds(base, CH)], sem_g, add=True)
            nxt = base + CH
            return jnp.where(nxt == TPW, 0, nxt)

        for _ in range(F):
            pltpu.make_async_copy(
                table_hbm.at[pl.ds(0, TPW)], emb_v, sem_g).wait()

        pltpu.sync_copy(emb_v, out_hbm.at[pl.ds(base_t, TPW)])

    return gather_kernel


def _mlp_body(v_ref, w1_ref, b1_ref, w2_ref, b2_ref, wo_ref, bo_ref,
              out_ref):
    v = v_ref[...]
    h1b = jnp.maximum(
        jnp.dot(v, w1_ref[...], preferred_element_type=jnp.float32)
        + b1_ref[...], 0.0)
    h1 = h1b.reshape(v.shape[0] * 8, w1_ref.shape[1] // 8)
    h2 = jnp.maximum(
        jnp.dot(h1, w2_ref[...], preferred_element_type=jnp.float32)
        + b2_ref[...], 0.0)
    o = jnp.dot(h2, wo_ref[...], preferred_element_type=jnp.float32) \
        + bo_ref[...]
    p = jax.nn.sigmoid(o)
    pc = jnp.concatenate([p[:, 0:1], p[:, 0:1] * p[:, 1:2]], axis=1)
    out_ref[...] = pc.T


@functools.lru_cache(maxsize=None)
def _make_mlp(T, D, H1c, H2c):
    BT = 3200 if T % 3200 == 0 else 2048
    grid = (T // BT,)

    def full(shape):
        return pl.BlockSpec(shape, lambda i: (0, 0))

    return pl.pallas_call(
        _mlp_body,
        grid=grid,
        in_specs=[
            pl.BlockSpec((BT // 8, 128), lambda i: (i, 0)),
            full((D * 8, H1c * 8)),
            full((1, H1c * 8)),
            full((H1c, H2c)),
            full((1, H2c)),
            full((H2c, 2)),
            full((1, 2)),
        ],
        out_specs=pl.BlockSpec((2, BT), lambda i: (0, i)),
        out_shape=jax.ShapeDtypeStruct((2, T), jnp.float32),
    )


def kernel(x, label_length, table,
           ctr_W1, ctr_b1, ctr_W2, ctr_b2, ctr_Wo, ctr_bo,
           cvr_W1, cvr_b1, cvr_W2, cvr_b2, cvr_Wo, cvr_bo):
    T, F = x.shape
    V1, D = table.shape
    H1 = ctr_W1.shape[1]
    H2 = ctr_W2.shape[1]

    table_fmt = _make_fmt_table(V1, D)(table.T)
    V_pad = table_fmt.shape[0] * 128 // D
    table_lin = table_fmt.reshape(V_pad, D)

    x_fmt = _make_fmt_x(F, T)(x.T)
    x_flat = x_fmt.reshape(F * T)

    NPART = 4
    TL = T // NPART

    z12 = jnp.zeros((H1, H2), jnp.float32)
    z2o = jnp.zeros((H2, 1), jnp.float32)
    W1 = jnp.concatenate([ctr_W1, cvr_W1], axis=1)
    b1 = jnp.concatenate([ctr_b1, cvr_b1])
    W1big = jnp.kron(jnp.eye(8, dtype=jnp.float32), W1)
    b1big = jnp.tile(b1, 8)[None, :]
    W2 = jnp.concatenate(
        [jnp.concatenate([ctr_W2, z12], axis=1),
         jnp.concatenate([z12, cvr_W2], axis=1)], axis=0)
    b2 = jnp.concatenate([ctr_b2, cvr_b2])[None, :]
    Wo = jnp.concatenate(
        [jnp.concatenate([ctr_Wo, z2o], axis=1),
         jnp.concatenate([z2o, cvr_Wo], axis=1)], axis=0)
    bo = jnp.concatenate([ctr_bo, cvr_bo])[None, :]

    outs = []
    for k in range(NPART):
        emb = _make_gather(T, F, D, V_pad, k * TL, TL)(x_flat, table_lin)
        emb2 = emb.reshape(TL * D // 128, 128)
        outs.append(_make_mlp(TL, D, 2 * H1, 2 * H2)(
            emb2, W1big, b1big, W2, b2, Wo, bo))
    return jnp.concatenate(outs, axis=1).T
